# Initial kernel scaffold; baseline (speedup 1.0000x reference)
import jax
import jax.numpy as jnp
from jax.experimental import pallas as pl


def kernel(x_pfas_sites, x_gw_wells, x_sw_stations, params, edge_index_pg, edge_index_gp, edge_index_ps, edge_index_sp):
    raise NotImplementedError("write your pallas kernel here")


# speedup vs baseline: 4.5298x; 4.5298x over previous
import functools

import jax
import jax.numpy as jnp
from jax import lax
from jax.experimental import pallas as pl
from jax.experimental.pallas import tpu as pltpu
from jax.experimental.pallas import tpu_sc as plsc

N = 10000
D = 128
DA = 144
E = 160000
NTILES = 16
EP = E // NTILES
K = 80
NC = EP // K
RPT0 = 632
RPTL = N - (NTILES - 1) * RPT0


def _make_seg_sum():
    mesh = plsc.VectorSubcoreMesh(core_axis_name="c", subcore_axis_name="s")
    out_type = [jax.ShapeDtypeStruct((N, DA), jnp.float32) for _ in range(4)]

    @functools.partial(
        pl.kernel,
        out_type=out_type,
        mesh=mesh,
        scratch_types=[
            pltpu.VMEM_SHARED((N, DA), jnp.float32),
            pltpu.VMEM((NC, K), jnp.int32),
            pltpu.VMEM((NC, K), jnp.int32),
            pltpu.VMEM((K, DA), jnp.float32),
            pltpu.SemaphoreType.DMA,
        ],
        compiler_params=pltpu.CompilerParams(use_tc_tiling_on_sc=False),
    )
    def seg_sum(t0, t1, t2, t3,
                s0, d0, s1, d1, s2, d2, s3, d3, zrows,
                o0, o1, o2, o3,
                acc, sidx, didx, rbuf, sem):
        c = lax.axis_index("c")
        s = lax.axis_index("s")

        def zero_slice():
            @pl.when(s < NTILES - 1)
            def _():
                pltpu.sync_copy(zrows, acc.at[pl.ds(s * RPT0, RPT0)])

            @pl.when(s == NTILES - 1)
            def _():
                pltpu.sync_copy(zrows.at[pl.ds(0, RPTL)],
                                acc.at[pl.ds((NTILES - 1) * RPT0, RPTL)])

        def process(tab, se, de):
            pltpu.sync_copy(se.at[s], sidx)
            pltpu.sync_copy(de.at[s], didx)

            def body(j, carry):
                pltpu.async_copy(tab.at[sidx.at[j]], rbuf, sem).wait()
                pltpu.sync_copy(rbuf, acc.at[didx.at[j]], add=True)
                return carry

            lax.fori_loop(0, NC, body, 0)

        def flush(out):
            @pl.when(s < NTILES - 1)
            def _():
                pltpu.sync_copy(acc.at[pl.ds(s * RPT0, RPT0)],
                                out.at[pl.ds(s * RPT0, RPT0)])

            @pl.when(s == NTILES - 1)
            def _():
                pltpu.sync_copy(acc.at[pl.ds((NTILES - 1) * RPT0, RPTL)],
                                out.at[pl.ds((NTILES - 1) * RPT0, RPTL)])

        def run_pair(tA, sA, dA, oA, tB, sB, dB, oB):
            zero_slice()
            plsc.subcore_barrier()
            process(tA, sA, dA)
            plsc.subcore_barrier()
            flush(oA)
            zero_slice()
            plsc.subcore_barrier()
            process(tB, sB, dB)
            plsc.subcore_barrier()
            flush(oB)

        @pl.when(c == 0)
        def _():
            run_pair(t0, s0, d0, o0, t1, s1, d1, o1)

        @pl.when(c == 1)
        def _():
            run_pair(t2, s2, d2, o2, t3, s3, d3, o3)

    return seg_sum


_seg_sum_cache = []


def _seg_sum(*args):
    if not _seg_sum_cache:
        _seg_sum_cache.append(_make_seg_sum())
    return _seg_sum_cache[0](*args)


BS = 2000
NB = N // BS


def _mm_body(xg_ref, xs_ref, xp_ref,
             mpg_ref, mgp_ref, mps_ref, msp_ref,
             cpg_ref, cgp_ref, cps_ref, csp_ref,
             wlpg, wrpg, bpg, wlgp, wrgp, bgp,
             wlps, wrps, bps, wlsp, wrsp, bsp,
             hg_ref, hs_ref, hp_ref, st_ref):
    f32 = jnp.float32
    i = pl.program_id(0)

    def sage(m_ref, c_ref, xd_ref, wl, wr, b):
        mean = m_ref[...] / jnp.maximum(c_ref[...], 1.0)
        return (jnp.dot(mean, wl[...], preferred_element_type=f32)
                + jnp.dot(xd_ref[...], wr[...], preferred_element_type=f32)
                + b[...])

    hg = sage(mpg_ref, cpg_ref, xg_ref, wlpg, wrpg, bpg)
    hs = sage(mps_ref, cps_ref, xs_ref, wlps, wrps, bps)
    hp = (sage(mgp_ref, cgp_ref, xp_ref, wlgp, wrgp, bgp)
          + sage(msp_ref, csp_ref, xp_ref, wlsp, wrsp, bsp))
    hg_ref[...] = hg
    hs_ref[...] = hs
    hp_ref[...] = hp

    z = jnp.zeros((1, D), f32)
    blk = jnp.concatenate(
        [jnp.sum(hg, axis=0, keepdims=True),
         jnp.sum(hg * hg, axis=0, keepdims=True),
         jnp.sum(hs, axis=0, keepdims=True),
         jnp.sum(hs * hs, axis=0, keepdims=True),
         z, z, z, z], axis=0)

    @pl.when(i == 0)
    def _():
        st_ref[...] = blk

    @pl.when(i > 0)
    def _():
        st_ref[...] = st_ref[...] + blk


def _bn_stats(st):
    mg = st[0:1, :] / N
    vg = st[1:2, :] / N - mg * mg
    ms = st[2:3, :] / N
    vs = st[3:4, :] / N - ms * ms
    return mg, vg, ms, vs


def _bn_body(hg_ref, hs_ref, st_ref, gg, gbg, gs, gbs, og_ref, os_ref):
    mg, vg, ms, vs = _bn_stats(st_ref[...])
    og_ref[...] = jax.nn.relu(
        (hg_ref[...] - mg) * jax.lax.rsqrt(vg + 1e-5) * gg[...] + gbg[...])
    os_ref[...] = jax.nn.relu(
        (hs_ref[...] - ms) * jax.lax.rsqrt(vs + 1e-5) * gs[...] + gbs[...])


def _head_body(hg_ref, hs_ref, st_ref, gg, gbg, gs, gbs,
               lwg, lbg, lws, lbs, ag, as_, outg_ref, outs_ref):
    mg, vg, ms, vs = _bn_stats(st_ref[...])
    og = jax.nn.relu(
        (hg_ref[...] - mg) * jax.lax.rsqrt(vg + 1e-5) * gg[...] + gbg[...])
    os_ = jax.nn.relu(
        (hs_ref[...] - ms) * jax.lax.rsqrt(vs + 1e-5) * gs[...] + gbs[...])

    def head(o, lw, lb, a):
        t = jnp.sum(o * lw[...], axis=1, keepdims=True) + lb[...]
        return jnp.where(t >= 0, t, a[...] * t)

    outg_ref[...] = head(og, lwg, lbg, ag)
    outs_ref[...] = head(os_, lws, lbs, as_)


def _row_spec(cols):
    return pl.BlockSpec((BS, cols), lambda i: (i, 0))


def _full_spec(r, c):
    return pl.BlockSpec((r, c), lambda i: (0, 0))


_W_SPECS = [_full_spec(D, D), _full_spec(D, D), _full_spec(1, D)] * 4
_BN_SPECS = [_full_spec(1, D)] * 4


def _dense_mm(xg, xs, xp, means, cnts, w, interpret=False):
    outs = (jax.ShapeDtypeStruct((N, D), jnp.float32),) * 3 \
        + (jax.ShapeDtypeStruct((8, D), jnp.float32),)
    in_specs = ([_row_spec(D)] * 3 + [_row_spec(D)] * 4 + [_row_spec(1)] * 4
                + _W_SPECS)
    out_specs = (_row_spec(D),) * 3 + (_full_spec(8, D),)
    return pl.pallas_call(
        _mm_body, grid=(NB,), out_shape=outs,
        in_specs=in_specs, out_specs=out_specs,
        interpret=interpret)(xg, xs, xp, *means, *cnts, *w)


def _dense_bn(hg, hs, st, bn, interpret=False):
    outs = (jax.ShapeDtypeStruct((N, D), jnp.float32),) * 2
    in_specs = [_row_spec(D)] * 2 + [_full_spec(8, D)] + _BN_SPECS
    out_specs = (_row_spec(D),) * 2
    return pl.pallas_call(
        _bn_body, grid=(NB,), out_shape=outs,
        in_specs=in_specs, out_specs=out_specs,
        interpret=interpret)(hg, hs, st, *bn)


def _dense_head(hg, hs, st, bn, lin, interpret=False):
    outs = (jax.ShapeDtypeStruct((N, 1), jnp.float32),) * 2
    in_specs = ([_row_spec(D)] * 2 + [_full_spec(8, D)] + _BN_SPECS
                + [_full_spec(1, D), _full_spec(1, 1)] * 2
                + [_full_spec(1, 1)] * 2)
    out_specs = (_row_spec(1),) * 2
    return pl.pallas_call(
        _head_body, grid=(NB,), out_shape=outs,
        in_specs=in_specs, out_specs=out_specs,
        interpret=interpret)(hg, hs, st, *bn, *lin)


def _augment(x):
    ones = jnp.ones((x.shape[0], 1), x.dtype)
    zpad = jnp.zeros((x.shape[0], DA - D - 1), x.dtype)
    return jnp.concatenate([x, ones, zpad], axis=1)


def _edges(ei):
    e = ei.astype(jnp.int32)
    return e[0].reshape(NTILES, NC, K), e[1].reshape(NTILES, NC, K)


def _sage_weights(params, tag):
    out = []
    for rel in ("pg", "gp", "ps", "sp"):
        p = params[f"{tag}_{rel}"]
        out += [p["Wl"], p["Wr"], p["b"].reshape(1, D)]
    return out


def kernel(x_pfas_sites, x_gw_wells, x_sw_stations, params,
           edge_index_pg, edge_index_gp, edge_index_ps, edge_index_sp):
    x_p, x_g, x_s = x_pfas_sites, x_gw_wells, x_sw_stations

    spg, dpg = _edges(edge_index_pg)
    sgp, dgp = _edges(edge_index_gp)
    sps, dps = _edges(edge_index_ps)
    ssp, dsp = _edges(edge_index_sp)
    zrows = jnp.zeros((RPT0, DA), jnp.float32)

    bn = [params["bn_gw"]["g"].reshape(1, D), params["bn_gw"]["b"].reshape(1, D),
          params["bn_sw"]["g"].reshape(1, D), params["bn_sw"]["b"].reshape(1, D)]

    def seg(tab_p, tab_g, tab_s):
        s1, s2, s3, s4 = _seg_sum(tab_p, tab_g, tab_p, tab_s,
                                  spg, dpg, sgp, dgp, sps, dps, ssp, dsp,
                                  zrows)
        means = (s1[:, :D], s2[:, :D], s3[:, :D], s4[:, :D])
        cnts = (s1[:, D:D + 1], s2[:, D:D + 1], s3[:, D:D + 1], s4[:, D:D + 1])
        return means, cnts

    means1, cnts1 = seg(_augment(x_p), _augment(x_g), _augment(x_s))
    hg1, hs1, h_p, st1 = _dense_mm(x_g, x_s, x_p, means1, cnts1,
                                   _sage_weights(params, "c1"))
    h_g, h_s = _dense_bn(hg1, hs1, st1, bn)

    means2, cnts2 = seg(_augment(h_p), _augment(h_g), _augment(h_s))
    og2, os2, o_p, st2 = _dense_mm(h_g, h_s, h_p, means2, cnts2,
                                   _sage_weights(params, "c2"))
    lin = [params["lin_gw"]["W"].reshape(1, D), params["lin_gw"]["b"].reshape(1, 1),
           params["lin_sw"]["W"].reshape(1, D), params["lin_sw"]["b"].reshape(1, 1),
           params["pr_gw"].reshape(1, 1), params["pr_sw"].reshape(1, 1)]
    out_g, out_s = _dense_head(og2, os2, st2, bn, lin)
    return out_g, out_s, o_p


# speedup vs baseline: 6.0754x; 1.3412x over previous
import functools

import jax
import jax.numpy as jnp
from jax import lax
from jax.experimental import pallas as pl
from jax.experimental.pallas import tpu as pltpu
from jax.experimental.pallas import tpu_sc as plsc

N = 10000
D = 128
DA = 144
E = 160000
NTILES = 16
EP = E // NTILES
K = 80
NC = EP // K
NCB = 25
RPT0 = 632
RPTL = N - (NTILES - 1) * RPT0


def _make_seg_sum():
    mesh = plsc.VectorSubcoreMesh(core_axis_name="c", subcore_axis_name="s")
    out_type = [jax.ShapeDtypeStruct((N, DA), jnp.float32) for _ in range(4)]

    @functools.partial(
        pl.kernel,
        out_type=out_type,
        mesh=mesh,
        scratch_types=[
            pltpu.VMEM_SHARED((N, DA), jnp.float32),
            pltpu.VMEM((NCB, K), jnp.int32),
            pltpu.VMEM((NCB, K), jnp.int32),
            pltpu.VMEM((2, K, DA), jnp.float32),
            pltpu.SemaphoreType.DMA,
            pltpu.SemaphoreType.DMA,
        ],
        compiler_params=pltpu.CompilerParams(use_tc_tiling_on_sc=False),
    )
    def seg_sum(t0, t1, t2, t3,
                s0, d0, s1, d1, s2, d2, s3, d3, zrows,
                o0, o1, o2, o3,
                acc, sidx, didx, rbuf, gsem, ssem):
        c = lax.axis_index("c")
        s = lax.axis_index("s")

        def zero_slice():
            @pl.when(s < NTILES - 1)
            def _():
                pltpu.sync_copy(zrows, acc.at[pl.ds(s * RPT0, RPT0)])

            @pl.when(s == NTILES - 1)
            def _():
                pltpu.sync_copy(zrows.at[pl.ds(0, RPTL)],
                                acc.at[pl.ds((NTILES - 1) * RPT0, RPTL)])

        def process(tab, se, de):
            pltpu.sync_copy(se.at[s, pl.ds(0, NCB)], sidx)
            pltpu.sync_copy(de.at[s, pl.ds(0, NCB)], didx)
            pltpu.async_copy(tab.at[sidx.at[0]], rbuf.at[0], gsem)

            def body(j, carry):
                b = lax.rem(j, 2)
                nb = 1 - b
                jm = lax.rem(j, NCB)
                jn = lax.rem(j + 1, NCB)

                @pl.when(jnp.logical_and(j >= 1, jm != 0))
                def _():
                    pltpu.make_async_copy(
                        rbuf.at[nb], acc.at[didx.at[lax.rem(j - 1, NCB)]],
                        ssem).wait()

                @pl.when(jnp.logical_and(j + 1 < NC, jn != 0))
                def _():
                    pltpu.async_copy(tab.at[sidx.at[jn]], rbuf.at[nb], gsem)

                pltpu.make_async_copy(
                    tab.at[sidx.at[jm]], rbuf.at[b], gsem).wait()
                pltpu.async_copy(rbuf.at[b], acc.at[didx.at[jm]], ssem,
                                 add=True)

                @pl.when(jnp.logical_and(j + 1 < NC, jn == 0))
                def _():
                    pltpu.make_async_copy(
                        rbuf.at[b], acc.at[didx.at[jm]], ssem).wait()
                    pltpu.sync_copy(se.at[s, pl.ds(j + 1, NCB)], sidx)
                    pltpu.sync_copy(de.at[s, pl.ds(j + 1, NCB)], didx)
                    pltpu.async_copy(tab.at[sidx.at[0]], rbuf.at[nb], gsem)
                return carry

            lax.fori_loop(0, NC, body, 0)
            pltpu.make_async_copy(
                rbuf.at[(NC - 1) % 2],
                acc.at[didx.at[(NC - 1) % NCB]], ssem).wait()

        def flush(out):
            @pl.when(s < NTILES - 1)
            def _():
                pltpu.sync_copy(acc.at[pl.ds(s * RPT0, RPT0)],
                                out.at[pl.ds(s * RPT0, RPT0)])

            @pl.when(s == NTILES - 1)
            def _():
                pltpu.sync_copy(acc.at[pl.ds((NTILES - 1) * RPT0, RPTL)],
                                out.at[pl.ds((NTILES - 1) * RPT0, RPTL)])

        def run_pair(tA, sA, dA, oA, tB, sB, dB, oB):
            zero_slice()
            plsc.subcore_barrier()
            process(tA, sA, dA)
            plsc.subcore_barrier()
            flush(oA)
            zero_slice()
            plsc.subcore_barrier()
            process(tB, sB, dB)
            plsc.subcore_barrier()
            flush(oB)

        @pl.when(c == 0)
        def _():
            run_pair(t0, s0, d0, o0, t1, s1, d1, o1)

        @pl.when(c == 1)
        def _():
            run_pair(t2, s2, d2, o2, t3, s3, d3, o3)

    return seg_sum


_seg_sum_cache = []


def _seg_sum(*args):
    if not _seg_sum_cache:
        _seg_sum_cache.append(_make_seg_sum())
    return _seg_sum_cache[0](*args)


BS = 2000
NB = N // BS


def _mm_body(xg_ref, xs_ref, xp_ref,
             mpg_ref, mgp_ref, mps_ref, msp_ref,
             cpg_ref, cgp_ref, cps_ref, csp_ref,
             wlpg, wrpg, bpg, wlgp, wrgp, bgp,
             wlps, wrps, bps, wlsp, wrsp, bsp,
             hg_ref, hs_ref, hp_ref, st_ref):
    f32 = jnp.float32
    i = pl.program_id(0)

    def sage(m_ref, c_ref, xd_ref, wl, wr, b):
        mean = m_ref[...] / jnp.maximum(c_ref[...], 1.0)
        return (jnp.dot(mean, wl[...], preferred_element_type=f32)
                + jnp.dot(xd_ref[...], wr[...], preferred_element_type=f32)
                + b[...])

    hg = sage(mpg_ref, cpg_ref, xg_ref, wlpg, wrpg, bpg)
    hs = sage(mps_ref, cps_ref, xs_ref, wlps, wrps, bps)
    hp = (sage(mgp_ref, cgp_ref, xp_ref, wlgp, wrgp, bgp)
          + sage(msp_ref, csp_ref, xp_ref, wlsp, wrsp, bsp))
    hg_ref[...] = hg
    hs_ref[...] = hs
    hp_ref[...] = hp

    z = jnp.zeros((1, D), f32)
    blk = jnp.concatenate(
        [jnp.sum(hg, axis=0, keepdims=True),
         jnp.sum(hg * hg, axis=0, keepdims=True),
         jnp.sum(hs, axis=0, keepdims=True),
         jnp.sum(hs * hs, axis=0, keepdims=True),
         z, z, z, z], axis=0)

    @pl.when(i == 0)
    def _():
        st_ref[...] = blk

    @pl.when(i > 0)
    def _():
        st_ref[...] = st_ref[...] + blk


def _bn_stats(st):
    mg = st[0:1, :] / N
    vg = st[1:2, :] / N - mg * mg
    ms = st[2:3, :] / N
    vs = st[3:4, :] / N - ms * ms
    return mg, vg, ms, vs


def _bn_body(hg_ref, hs_ref, st_ref, gg, gbg, gs, gbs, og_ref, os_ref):
    mg, vg, ms, vs = _bn_stats(st_ref[...])
    og_ref[...] = jax.nn.relu(
        (hg_ref[...] - mg) * jax.lax.rsqrt(vg + 1e-5) * gg[...] + gbg[...])
    os_ref[...] = jax.nn.relu(
        (hs_ref[...] - ms) * jax.lax.rsqrt(vs + 1e-5) * gs[...] + gbs[...])


def _head_body(hg_ref, hs_ref, st_ref, gg, gbg, gs, gbs,
               lwg, lbg, lws, lbs, ag, as_, outg_ref, outs_ref):
    mg, vg, ms, vs = _bn_stats(st_ref[...])
    og = jax.nn.relu(
        (hg_ref[...] - mg) * jax.lax.rsqrt(vg + 1e-5) * gg[...] + gbg[...])
    os_ = jax.nn.relu(
        (hs_ref[...] - ms) * jax.lax.rsqrt(vs + 1e-5) * gs[...] + gbs[...])

    def head(o, lw, lb, a):
        t = jnp.sum(o * lw[...], axis=1, keepdims=True) + lb[...]
        return jnp.where(t >= 0, t, a[...] * t)

    outg_ref[...] = head(og, lwg, lbg, ag)
    outs_ref[...] = head(os_, lws, lbs, as_)


def _row_spec(cols):
    return pl.BlockSpec((BS, cols), lambda i: (i, 0))


def _full_spec(r, c):
    return pl.BlockSpec((r, c), lambda i: (0, 0))


_W_SPECS = [_full_spec(D, D), _full_spec(D, D), _full_spec(1, D)] * 4
_BN_SPECS = [_full_spec(1, D)] * 4


def _dense_mm(xg, xs, xp, means, cnts, w, interpret=False):
    outs = (jax.ShapeDtypeStruct((N, D), jnp.float32),) * 3 \
        + (jax.ShapeDtypeStruct((8, D), jnp.float32),)
    in_specs = ([_row_spec(D)] * 3 + [_row_spec(D)] * 4 + [_row_spec(1)] * 4
                + _W_SPECS)
    out_specs = (_row_spec(D),) * 3 + (_full_spec(8, D),)
    return pl.pallas_call(
        _mm_body, grid=(NB,), out_shape=outs,
        in_specs=in_specs, out_specs=out_specs,
        interpret=interpret)(xg, xs, xp, *means, *cnts, *w)


def _dense_bn(hg, hs, st, bn, interpret=False):
    outs = (jax.ShapeDtypeStruct((N, D), jnp.float32),) * 2
    in_specs = [_row_spec(D)] * 2 + [_full_spec(8, D)] + _BN_SPECS
    out_specs = (_row_spec(D),) * 2
    return pl.pallas_call(
        _bn_body, grid=(NB,), out_shape=outs,
        in_specs=in_specs, out_specs=out_specs,
        interpret=interpret)(hg, hs, st, *bn)


def _dense_head(hg, hs, st, bn, lin, interpret=False):
    outs = (jax.ShapeDtypeStruct((N, 1), jnp.float32),) * 2
    in_specs = ([_row_spec(D)] * 2 + [_full_spec(8, D)] + _BN_SPECS
                + [_full_spec(1, D), _full_spec(1, 1)] * 2
                + [_full_spec(1, 1)] * 2)
    out_specs = (_row_spec(1),) * 2
    return pl.pallas_call(
        _head_body, grid=(NB,), out_shape=outs,
        in_specs=in_specs, out_specs=out_specs,
        interpret=interpret)(hg, hs, st, *bn, *lin)


def _augment(x):
    ones = jnp.ones((x.shape[0], 1), x.dtype)
    zpad = jnp.zeros((x.shape[0], DA - D - 1), x.dtype)
    return jnp.concatenate([x, ones, zpad], axis=1)


def _edges(ei):
    e = ei.astype(jnp.int32)
    return e[0].reshape(NTILES, NC, K), e[1].reshape(NTILES, NC, K)


def _sage_weights(params, tag):
    out = []
    for rel in ("pg", "gp", "ps", "sp"):
        p = params[f"{tag}_{rel}"]
        out += [p["Wl"], p["Wr"], p["b"].reshape(1, D)]
    return out


def kernel(x_pfas_sites, x_gw_wells, x_sw_stations, params,
           edge_index_pg, edge_index_gp, edge_index_ps, edge_index_sp):
    x_p, x_g, x_s = x_pfas_sites, x_gw_wells, x_sw_stations

    spg, dpg = _edges(edge_index_pg)
    sgp, dgp = _edges(edge_index_gp)
    sps, dps = _edges(edge_index_ps)
    ssp, dsp = _edges(edge_index_sp)
    zrows = jnp.zeros((RPT0, DA), jnp.float32)

    bn = [params["bn_gw"]["g"].reshape(1, D), params["bn_gw"]["b"].reshape(1, D),
          params["bn_sw"]["g"].reshape(1, D), params["bn_sw"]["b"].reshape(1, D)]

    def seg(tab_p, tab_g, tab_s):
        s1, s2, s3, s4 = _seg_sum(tab_p, tab_g, tab_p, tab_s,
                                  spg, dpg, sgp, dgp, sps, dps, ssp, dsp,
                                  zrows)
        means = (s1[:, :D], s2[:, :D], s3[:, :D], s4[:, :D])
        cnts = (s1[:, D:D + 1], s2[:, D:D + 1], s3[:, D:D + 1], s4[:, D:D + 1])
        return means, cnts

    means1, cnts1 = seg(_augment(x_p), _augment(x_g), _augment(x_s))
    hg1, hs1, h_p, st1 = _dense_mm(x_g, x_s, x_p, means1, cnts1,
                                   _sage_weights(params, "c1"))
    h_g, h_s = _dense_bn(hg1, hs1, st1, bn)

    means2, cnts2 = seg(_augment(h_p), _augment(h_g), _augment(h_s))
    og2, os2, o_p, st2 = _dense_mm(h_g, h_s, h_p, means2, cnts2,
                                   _sage_weights(params, "c2"))
    lin = [params["lin_gw"]["W"].reshape(1, D), params["lin_gw"]["b"].reshape(1, 1),
           params["lin_sw"]["W"].reshape(1, D), params["lin_sw"]["b"].reshape(1, 1),
           params["pr_gw"].reshape(1, 1), params["pr_sw"].reshape(1, 1)]
    out_g, out_s = _dense_head(og2, os2, st2, bn, lin)
    return out_g, out_s, o_p


# speedup vs baseline: 7.5941x; 1.2500x over previous
import functools

import jax
import jax.numpy as jnp
from jax import lax
from jax.experimental import pallas as pl
from jax.experimental.pallas import tpu as pltpu
from jax.experimental.pallas import tpu_sc as plsc

N = 10000
D = 128
DA = 144
E = 160000
NTILES = 16
EP = E // NTILES
K = 125
NC = EP // K
NCB = 10
RPT0 = 632
RPTL = N - (NTILES - 1) * RPT0


def _make_seg_sum(W):
    mesh = plsc.VectorSubcoreMesh(core_axis_name="c", subcore_axis_name="s")
    out_type = [jax.ShapeDtypeStruct((N, W), jnp.float32) for _ in range(4)]

    @functools.partial(
        pl.kernel,
        out_type=out_type,
        mesh=mesh,
        scratch_types=[
            pltpu.VMEM_SHARED((N, W), jnp.float32),
            pltpu.VMEM((NCB, K), jnp.int32),
            pltpu.VMEM((NCB, K), jnp.int32),
            pltpu.VMEM((2, K, W), jnp.float32),
            pltpu.SemaphoreType.DMA,
            pltpu.SemaphoreType.DMA,
        ],
        compiler_params=pltpu.CompilerParams(use_tc_tiling_on_sc=False),
    )
    def seg_sum(t0, t1, t2, t3,
                s0, d0, s1, d1, s2, d2, s3, d3, zrows,
                o0, o1, o2, o3,
                acc, sidx, didx, rbuf, gsem, ssem):
        c = lax.axis_index("c")
        s = lax.axis_index("s")

        def zero_slice():
            @pl.when(s < NTILES - 1)
            def _():
                pltpu.sync_copy(zrows, acc.at[pl.ds(s * RPT0, RPT0)])

            @pl.when(s == NTILES - 1)
            def _():
                pltpu.sync_copy(zrows.at[pl.ds(0, RPTL)],
                                acc.at[pl.ds((NTILES - 1) * RPT0, RPTL)])

        def process(tab, se, de):
            pltpu.sync_copy(se.at[s, pl.ds(0, NCB)], sidx)
            pltpu.sync_copy(de.at[s, pl.ds(0, NCB)], didx)
            pltpu.async_copy(tab.at[sidx.at[0]], rbuf.at[0], gsem)

            def body(j, carry):
                b = lax.rem(j, 2)
                nb = 1 - b
                jm = lax.rem(j, NCB)
                jn = lax.rem(j + 1, NCB)

                @pl.when(jnp.logical_and(j >= 1, jm != 0))
                def _():
                    pltpu.make_async_copy(
                        rbuf.at[nb], acc.at[didx.at[lax.rem(j - 1, NCB)]],
                        ssem).wait()

                @pl.when(jnp.logical_and(j + 1 < NC, jn != 0))
                def _():
                    pltpu.async_copy(tab.at[sidx.at[jn]], rbuf.at[nb], gsem)

                pltpu.make_async_copy(
                    tab.at[sidx.at[jm]], rbuf.at[b], gsem).wait()
                pltpu.async_copy(rbuf.at[b], acc.at[didx.at[jm]], ssem,
                                 add=True)

                @pl.when(jnp.logical_and(j + 1 < NC, jn == 0))
                def _():
                    pltpu.make_async_copy(
                        rbuf.at[b], acc.at[didx.at[jm]], ssem).wait()
                    pltpu.sync_copy(se.at[s, pl.ds(j + 1, NCB)], sidx)
                    pltpu.sync_copy(de.at[s, pl.ds(j + 1, NCB)], didx)
                    pltpu.async_copy(tab.at[sidx.at[0]], rbuf.at[nb], gsem)
                return carry

            lax.fori_loop(0, NC, body, 0)
            pltpu.make_async_copy(
                rbuf.at[(NC - 1) % 2],
                acc.at[didx.at[(NC - 1) % NCB]], ssem).wait()

        def flush(out):
            @pl.when(s < NTILES - 1)
            def _():
                pltpu.sync_copy(acc.at[pl.ds(s * RPT0, RPT0)],
                                out.at[pl.ds(s * RPT0, RPT0)])

            @pl.when(s == NTILES - 1)
            def _():
                pltpu.sync_copy(acc.at[pl.ds((NTILES - 1) * RPT0, RPTL)],
                                out.at[pl.ds((NTILES - 1) * RPT0, RPTL)])

        def run_pair(tA, sA, dA, oA, tB, sB, dB, oB):
            zero_slice()
            plsc.subcore_barrier()
            process(tA, sA, dA)
            plsc.subcore_barrier()
            flush(oA)
            zero_slice()
            plsc.subcore_barrier()
            process(tB, sB, dB)
            plsc.subcore_barrier()
            flush(oB)

        @pl.when(c == 0)
        def _():
            run_pair(t0, s0, d0, o0, t1, s1, d1, o1)

        @pl.when(c == 1)
        def _():
            run_pair(t2, s2, d2, o2, t3, s3, d3, o3)

    return seg_sum


_seg_sum_cache = {}


def _seg_sum(W, *args):
    if W not in _seg_sum_cache:
        _seg_sum_cache[W] = _make_seg_sum(W)
    return _seg_sum_cache[W](*args)


BS = 2000
NB = N // BS


def _mm_body(xg_ref, xs_ref, xp_ref,
             mpg_ref, mgp_ref, mps_ref, msp_ref,
             cpg_ref, cgp_ref, cps_ref, csp_ref,
             wlpg, wrpg, bpg, wlgp, wrgp, bgp,
             wlps, wrps, bps, wlsp, wrsp, bsp,
             hg_ref, hs_ref, hp_ref, st_ref):
    f32 = jnp.float32
    i = pl.program_id(0)

    def sage(m_ref, c_ref, xd_ref, wl, wr, b):
        mean = m_ref[...][:, :D] / jnp.maximum(c_ref[...], 1.0)
        return (jnp.dot(mean, wl[...], preferred_element_type=f32)
                + jnp.dot(xd_ref[...], wr[...], preferred_element_type=f32)
                + b[...])

    hg = sage(mpg_ref, cpg_ref, xg_ref, wlpg, wrpg, bpg)
    hs = sage(mps_ref, cps_ref, xs_ref, wlps, wrps, bps)
    hp = (sage(mgp_ref, cgp_ref, xp_ref, wlgp, wrgp, bgp)
          + sage(msp_ref, csp_ref, xp_ref, wlsp, wrsp, bsp))
    hg_ref[...] = hg
    hs_ref[...] = hs
    hp_ref[...] = hp

    z = jnp.zeros((1, D), f32)
    blk = jnp.concatenate(
        [jnp.sum(hg, axis=0, keepdims=True),
         jnp.sum(hg * hg, axis=0, keepdims=True),
         jnp.sum(hs, axis=0, keepdims=True),
         jnp.sum(hs * hs, axis=0, keepdims=True),
         z, z, z, z], axis=0)

    @pl.when(i == 0)
    def _():
        st_ref[...] = blk

    @pl.when(i > 0)
    def _():
        st_ref[...] = st_ref[...] + blk


def _bn_stats(st):
    mg = st[0:1, :] / N
    vg = st[1:2, :] / N - mg * mg
    ms = st[2:3, :] / N
    vs = st[3:4, :] / N - ms * ms
    return mg, vg, ms, vs


def _bn_body(hg_ref, hs_ref, st_ref, gg, gbg, gs, gbs, og_ref, os_ref):
    mg, vg, ms, vs = _bn_stats(st_ref[...])
    og_ref[...] = jax.nn.relu(
        (hg_ref[...] - mg) * jax.lax.rsqrt(vg + 1e-5) * gg[...] + gbg[...])
    os_ref[...] = jax.nn.relu(
        (hs_ref[...] - ms) * jax.lax.rsqrt(vs + 1e-5) * gs[...] + gbs[...])


def _head_body(hg_ref, hs_ref, st_ref, gg, gbg, gs, gbs,
               lwg, lbg, lws, lbs, ag, as_, outg_ref, outs_ref):
    mg, vg, ms, vs = _bn_stats(st_ref[...])
    og = jax.nn.relu(
        (hg_ref[...] - mg) * jax.lax.rsqrt(vg + 1e-5) * gg[...] + gbg[...])
    os_ = jax.nn.relu(
        (hs_ref[...] - ms) * jax.lax.rsqrt(vs + 1e-5) * gs[...] + gbs[...])

    def head(o, lw, lb, a):
        t = jnp.sum(o * lw[...], axis=1, keepdims=True) + lb[...]
        return jnp.where(t >= 0, t, a[...] * t)

    outg_ref[...] = head(og, lwg, lbg, ag)
    outs_ref[...] = head(os_, lws, lbs, as_)


def _row_spec(cols):
    return pl.BlockSpec((BS, cols), lambda i: (i, 0))


def _full_spec(r, c):
    return pl.BlockSpec((r, c), lambda i: (0, 0))


_W_SPECS = [_full_spec(D, D), _full_spec(D, D), _full_spec(1, D)] * 4
_BN_SPECS = [_full_spec(1, D)] * 4


def _dense_mm(xg, xs, xp, means, cnts, w, ws=D, interpret=False):
    outs = (jax.ShapeDtypeStruct((N, D), jnp.float32),) * 3 \
        + (jax.ShapeDtypeStruct((8, D), jnp.float32),)
    in_specs = ([_row_spec(D)] * 3 + [_row_spec(ws)] * 4 + [_row_spec(1)] * 4
                + _W_SPECS)
    out_specs = (_row_spec(D),) * 3 + (_full_spec(8, D),)
    return pl.pallas_call(
        _mm_body, grid=(NB,), out_shape=outs,
        in_specs=in_specs, out_specs=out_specs,
        interpret=interpret)(xg, xs, xp, *means, *cnts, *w)


def _dense_bn(hg, hs, st, bn, interpret=False):
    outs = (jax.ShapeDtypeStruct((N, D), jnp.float32),) * 2
    in_specs = [_row_spec(D)] * 2 + [_full_spec(8, D)] + _BN_SPECS
    out_specs = (_row_spec(D),) * 2
    return pl.pallas_call(
        _bn_body, grid=(NB,), out_shape=outs,
        in_specs=in_specs, out_specs=out_specs,
        interpret=interpret)(hg, hs, st, *bn)


def _dense_head(hg, hs, st, bn, lin, interpret=False):
    outs = (jax.ShapeDtypeStruct((N, 1), jnp.float32),) * 2
    in_specs = ([_row_spec(D)] * 2 + [_full_spec(8, D)] + _BN_SPECS
                + [_full_spec(1, D), _full_spec(1, 1)] * 2
                + [_full_spec(1, 1)] * 2)
    out_specs = (_row_spec(1),) * 2
    return pl.pallas_call(
        _head_body, grid=(NB,), out_shape=outs,
        in_specs=in_specs, out_specs=out_specs,
        interpret=interpret)(hg, hs, st, *bn, *lin)


def _augment(x):
    ones = jnp.ones((x.shape[0], 1), x.dtype)
    zpad = jnp.zeros((x.shape[0], DA - D - 1), x.dtype)
    return jnp.concatenate([x, ones, zpad], axis=1)


def _edges(ei):
    e = ei.astype(jnp.int32)
    return e[0].reshape(NTILES, NC, K), e[1].reshape(NTILES, NC, K)


def _sage_weights(params, tag):
    out = []
    for rel in ("pg", "gp", "ps", "sp"):
        p = params[f"{tag}_{rel}"]
        out += [p["Wl"], p["Wr"], p["b"].reshape(1, D)]
    return out


def kernel(x_pfas_sites, x_gw_wells, x_sw_stations, params,
           edge_index_pg, edge_index_gp, edge_index_ps, edge_index_sp):
    x_p, x_g, x_s = x_pfas_sites, x_gw_wells, x_sw_stations

    spg, dpg = _edges(edge_index_pg)
    sgp, dgp = _edges(edge_index_gp)
    sps, dps = _edges(edge_index_ps)
    ssp, dsp = _edges(edge_index_sp)
    zrows = jnp.zeros((RPT0, DA), jnp.float32)

    bn = [params["bn_gw"]["g"].reshape(1, D), params["bn_gw"]["b"].reshape(1, D),
          params["bn_sw"]["g"].reshape(1, D), params["bn_sw"]["b"].reshape(1, D)]

    def seg(W, tab_p, tab_g, tab_s, zr):
        return _seg_sum(W, tab_p, tab_g, tab_p, tab_s,
                        spg, dpg, sgp, dgp, sps, dps, ssp, dsp, zr)

    s1 = seg(DA, _augment(x_p), _augment(x_g), _augment(x_s), zrows)
    cnts = tuple(t[:, D:D + 1] for t in s1)
    hg1, hs1, h_p, st1 = _dense_mm(x_g, x_s, x_p, s1, cnts,
                                   _sage_weights(params, "c1"), ws=DA)
    h_g, h_s = _dense_bn(hg1, hs1, st1, bn)

    s2 = seg(D, h_p, h_g, h_s, zrows[:, :D])
    og2, os2, o_p, st2 = _dense_mm(h_g, h_s, h_p, s2, cnts,
                                   _sage_weights(params, "c2"), ws=D)
    lin = [params["lin_gw"]["W"].reshape(1, D), params["lin_gw"]["b"].reshape(1, 1),
           params["lin_sw"]["W"].reshape(1, D), params["lin_sw"]["b"].reshape(1, 1),
           params["pr_gw"].reshape(1, 1), params["pr_sw"].reshape(1, 1)]
    out_g, out_s = _dense_head(og2, os2, st2, bn, lin)
    return out_g, out_s, o_p


# speedup vs baseline: 7.6313x; 1.0049x over previous
import functools

import jax
import jax.numpy as jnp
from jax import lax
from jax.experimental import pallas as pl
from jax.experimental.pallas import tpu as pltpu
from jax.experimental.pallas import tpu_sc as plsc

N = 10000
D = 128
DA = 144
E = 160000
NTILES = 16
EP = E // NTILES
K = 125
NC = EP // K
NCB = 10
RPT0 = 632
RPTL = N - (NTILES - 1) * RPT0


def _make_seg_sum(W):
    mesh = plsc.VectorSubcoreMesh(core_axis_name="c", subcore_axis_name="s")
    out_type = [jax.ShapeDtypeStruct((N, W), jnp.float32) for _ in range(4)]

    @functools.partial(
        pl.kernel,
        out_type=out_type,
        mesh=mesh,
        scratch_types=[
            pltpu.VMEM_SHARED((N, W), jnp.float32),
            pltpu.VMEM((NCB, K), jnp.int32),
            pltpu.VMEM((NCB, K), jnp.int32),
            pltpu.VMEM((2, K, W), jnp.float32),
            pltpu.SemaphoreType.DMA,
            pltpu.SemaphoreType.DMA,
        ],
        compiler_params=pltpu.CompilerParams(use_tc_tiling_on_sc=False),
    )
    def seg_sum(t0, t1, t2, t3,
                s0, d0, s1, d1, s2, d2, s3, d3, zrows,
                o0, o1, o2, o3,
                acc, sidx, didx, rbuf, gsem, ssem):
        c = lax.axis_index("c")
        s = lax.axis_index("s")

        def zero_slice():
            @pl.when(s < NTILES - 1)
            def _():
                pltpu.sync_copy(zrows, acc.at[pl.ds(s * RPT0, RPT0)])

            @pl.when(s == NTILES - 1)
            def _():
                pltpu.sync_copy(zrows.at[pl.ds(0, RPTL)],
                                acc.at[pl.ds((NTILES - 1) * RPT0, RPTL)])

        def process(tab, se, de):
            pltpu.sync_copy(se.at[s, pl.ds(0, NCB)], sidx)
            pltpu.sync_copy(de.at[s, pl.ds(0, NCB)], didx)
            pltpu.async_copy(tab.at[sidx.at[0]], rbuf.at[0], gsem)

            def body(j, carry):
                b = lax.rem(j, 2)
                nb = 1 - b
                jm = lax.rem(j, NCB)
                jn = lax.rem(j + 1, NCB)

                @pl.when(jnp.logical_and(j >= 1, jm != 0))
                def _():
                    pltpu.make_async_copy(
                        rbuf.at[nb], acc.at[didx.at[lax.rem(j - 1, NCB)]],
                        ssem).wait()

                @pl.when(jnp.logical_and(j + 1 < NC, jn != 0))
                def _():
                    pltpu.async_copy(tab.at[sidx.at[jn]], rbuf.at[nb], gsem)

                pltpu.make_async_copy(
                    tab.at[sidx.at[jm]], rbuf.at[b], gsem).wait()
                pltpu.async_copy(rbuf.at[b], acc.at[didx.at[jm]], ssem,
                                 add=True)

                @pl.when(jnp.logical_and(j + 1 < NC, jn == 0))
                def _():
                    pltpu.make_async_copy(
                        rbuf.at[b], acc.at[didx.at[jm]], ssem).wait()
                    pltpu.sync_copy(se.at[s, pl.ds(j + 1, NCB)], sidx)
                    pltpu.sync_copy(de.at[s, pl.ds(j + 1, NCB)], didx)
                    pltpu.async_copy(tab.at[sidx.at[0]], rbuf.at[nb], gsem)
                return carry

            lax.fori_loop(0, NC, body, 0)
            pltpu.make_async_copy(
                rbuf.at[(NC - 1) % 2],
                acc.at[didx.at[(NC - 1) % NCB]], ssem).wait()

        def flush(out):
            @pl.when(s < NTILES - 1)
            def _():
                pltpu.sync_copy(acc.at[pl.ds(s * RPT0, RPT0)],
                                out.at[pl.ds(s * RPT0, RPT0)])

            @pl.when(s == NTILES - 1)
            def _():
                pltpu.sync_copy(acc.at[pl.ds((NTILES - 1) * RPT0, RPTL)],
                                out.at[pl.ds((NTILES - 1) * RPT0, RPTL)])

        def run_pair(tA, sA, dA, oA, tB, sB, dB, oB):
            zero_slice()
            plsc.subcore_barrier()
            process(tA, sA, dA)
            plsc.subcore_barrier()
            flush(oA)
            zero_slice()
            plsc.subcore_barrier()
            process(tB, sB, dB)
            plsc.subcore_barrier()
            flush(oB)

        @pl.when(c == 0)
        def _():
            run_pair(t0, s0, d0, o0, t1, s1, d1, o1)

        @pl.when(c == 1)
        def _():
            run_pair(t2, s2, d2, o2, t3, s3, d3, o3)

    return seg_sum


_seg_sum_cache = {}


def _seg_sum(W, *args):
    if W not in _seg_sum_cache:
        _seg_sum_cache[W] = _make_seg_sum(W)
    return _seg_sum_cache[W](*args)


BS = 2000
NB = N // BS


def _mmr_body(xg_ref, xs_ref, xp_ref,
              wrpg, bpg, wrgp, bgp, wrps, bps, wrsp, bsp,
              rg_ref, rs_ref, rp_ref):
    f32 = jnp.float32
    rg_ref[...] = jnp.dot(xg_ref[...], wrpg[...],
                          preferred_element_type=f32) + bpg[...]
    rs_ref[...] = jnp.dot(xs_ref[...], wrps[...],
                          preferred_element_type=f32) + bps[...]
    rp_ref[...] = (jnp.dot(xp_ref[...], wrgp[...],
                           preferred_element_type=f32) + bgp[...]
                   + jnp.dot(xp_ref[...], wrsp[...],
                             preferred_element_type=f32) + bsp[...])


def _dense_mmr(xg, xs, xp, w, interpret=False):
    outs = (jax.ShapeDtypeStruct((N, D), jnp.float32),) * 3
    row = pl.BlockSpec((BS, D), lambda i: (i, 0))
    in_specs = ([row] * 3
                + [pl.BlockSpec((D, D), lambda i: (0, 0)),
                   pl.BlockSpec((1, D), lambda i: (0, 0))] * 4)
    return pl.pallas_call(
        _mmr_body, grid=(NB,), out_shape=outs,
        in_specs=in_specs, out_specs=(row,) * 3,
        interpret=interpret)(xg, xs, xp, *w)


def _bn_stats(st):
    mg = st[0:1, :] / N
    vg = st[1:2, :] / N - mg * mg
    ms = st[2:3, :] / N
    vs = st[3:4, :] / N - ms * ms
    return mg, vg, ms, vs


def _ml_phase0(spg_ref, sgp_ref, sps_ref, ssp_ref,
               cpg_ref, cgp_ref, cps_ref, csp_ref,
               rg_ref, rs_ref, rp_ref,
               wlpg, wlgp, wlps, wlsp,
               hp_ref, pre_g, pre_s, sts, i):
    f32 = jnp.float32

    def ml(s_ref, c_ref, wl):
        mean = s_ref[...][:, :D] / jnp.maximum(c_ref[...], 1.0)
        return jnp.dot(mean, wl[...], preferred_element_type=f32)

    hg = ml(spg_ref, cpg_ref, wlpg) + rg_ref[...]
    hs = ml(sps_ref, cps_ref, wlps) + rs_ref[...]
    hp_ref[...] = (ml(sgp_ref, cgp_ref, wlgp)
                   + ml(ssp_ref, csp_ref, wlsp) + rp_ref[...])
    pre_g[i] = hg
    pre_s[i] = hs

    z = jnp.zeros((1, D), f32)
    blk = jnp.concatenate(
        [jnp.sum(hg, axis=0, keepdims=True),
         jnp.sum(hg * hg, axis=0, keepdims=True),
         jnp.sum(hs, axis=0, keepdims=True),
         jnp.sum(hs * hs, axis=0, keepdims=True),
         z, z, z, z], axis=0)

    @pl.when(i == 0)
    def _():
        sts[...] = blk

    @pl.when(i > 0)
    def _():
        sts[...] = sts[...] + blk


def _ml_bn_body(spg_ref, sgp_ref, sps_ref, ssp_ref,
                cpg_ref, cgp_ref, cps_ref, csp_ref,
                rg_ref, rs_ref, rp_ref,
                wlpg, wlgp, wlps, wlsp,
                gg, gbg, gs, gbs,
                hg_ref, hs_ref, hp_ref,
                pre_g, pre_s, sts):
    ph = pl.program_id(0)
    i = pl.program_id(1)

    @pl.when(ph == 0)
    def _():
        _ml_phase0(spg_ref, sgp_ref, sps_ref, ssp_ref,
                   cpg_ref, cgp_ref, cps_ref, csp_ref,
                   rg_ref, rs_ref, rp_ref,
                   wlpg, wlgp, wlps, wlsp,
                   hp_ref, pre_g, pre_s, sts, i)

    @pl.when(ph == 1)
    def _():
        mg, vg, ms, vs = _bn_stats(sts[...])
        hg_ref[...] = jax.nn.relu(
            (pre_g[i] - mg) * jax.lax.rsqrt(vg + 1e-5) * gg[...] + gbg[...])
        hs_ref[...] = jax.nn.relu(
            (pre_s[i] - ms) * jax.lax.rsqrt(vs + 1e-5) * gs[...] + gbs[...])


def _ml_head_body(spg_ref, sgp_ref, sps_ref, ssp_ref,
                  cpg_ref, cgp_ref, cps_ref, csp_ref,
                  rg_ref, rs_ref, rp_ref,
                  wlpg, wlgp, wlps, wlsp,
                  gg, gbg, gs, gbs,
                  lwg, lbg, lws, lbs, ag, as_,
                  outg_ref, outs_ref, op_ref,
                  pre_g, pre_s, sts):
    ph = pl.program_id(0)
    i = pl.program_id(1)

    @pl.when(ph == 0)
    def _():
        _ml_phase0(spg_ref, sgp_ref, sps_ref, ssp_ref,
                   cpg_ref, cgp_ref, cps_ref, csp_ref,
                   rg_ref, rs_ref, rp_ref,
                   wlpg, wlgp, wlps, wlsp,
                   op_ref, pre_g, pre_s, sts, i)

    @pl.when(ph == 1)
    def _():
        mg, vg, ms, vs = _bn_stats(sts[...])
        og = jax.nn.relu(
            (pre_g[i] - mg) * jax.lax.rsqrt(vg + 1e-5) * gg[...] + gbg[...])
        os_ = jax.nn.relu(
            (pre_s[i] - ms) * jax.lax.rsqrt(vs + 1e-5) * gs[...] + gbs[...])

        def head(o, lw, lb, a):
            t = jnp.sum(o * lw[...], axis=1, keepdims=True) + lb[...]
            return jnp.where(t >= 0, t, a[...] * t)

        outg_ref[...] = head(og, lwg, lbg, ag)
        outs_ref[...] = head(os_, lws, lbs, as_)


def _p0_spec(cols):
    return pl.BlockSpec(
        (BS, cols), lambda ph, i: (jnp.where(ph == 0, i, 0), 0))


def _p1out_spec(cols):
    return pl.BlockSpec(
        (BS, cols), lambda ph, i: (jnp.where(ph == 0, 0, i), 0))


def _p0out_spec(cols):
    return pl.BlockSpec(
        (BS, cols), lambda ph, i: (jnp.where(ph == 0, i, NB - 1), 0))


def _pfull_spec(r, c):
    return pl.BlockSpec((r, c), lambda ph, i: (0, 0))


def _ml_scratch():
    return [pltpu.VMEM((NB, BS, D), jnp.float32),
            pltpu.VMEM((NB, BS, D), jnp.float32),
            pltpu.VMEM((8, D), jnp.float32)]


def _dense_ml_bn(s, cnts, r, wl, bn, ws=D, interpret=False):
    outs = (jax.ShapeDtypeStruct((N, D), jnp.float32),) * 3
    in_specs = ([_p0_spec(ws)] * 4 + [_p0_spec(1)] * 4 + [_p0_spec(D)] * 3
                + [_pfull_spec(D, D)] * 4 + [_pfull_spec(1, D)] * 4)
    out_specs = (_p1out_spec(D), _p1out_spec(D), _p0out_spec(D))
    return pl.pallas_call(
        _ml_bn_body, grid=(2, NB), out_shape=outs,
        in_specs=in_specs, out_specs=out_specs,
        scratch_shapes=_ml_scratch(),
        interpret=interpret)(*s, *cnts, *r, *wl, *bn)


def _dense_ml_head(s, cnts, r, wl, bn, lin, ws=D, interpret=False):
    outs = (jax.ShapeDtypeStruct((N, 1), jnp.float32),
            jax.ShapeDtypeStruct((N, 1), jnp.float32),
            jax.ShapeDtypeStruct((N, D), jnp.float32))
    in_specs = ([_p0_spec(ws)] * 4 + [_p0_spec(1)] * 4 + [_p0_spec(D)] * 3
                + [_pfull_spec(D, D)] * 4 + [_pfull_spec(1, D)] * 4
                + [_pfull_spec(1, D), _pfull_spec(1, 1)] * 2
                + [_pfull_spec(1, 1)] * 2)
    out_specs = (_p1out_spec(1), _p1out_spec(1), _p0out_spec(D))
    return pl.pallas_call(
        _ml_head_body, grid=(2, NB), out_shape=outs,
        in_specs=in_specs, out_specs=out_specs,
        scratch_shapes=_ml_scratch(),
        interpret=interpret)(*s, *cnts, *r, *wl, *bn, *lin)


def _augment(x):
    ones = jnp.ones((x.shape[0], 1), x.dtype)
    zpad = jnp.zeros((x.shape[0], DA - D - 1), x.dtype)
    return jnp.concatenate([x, ones, zpad], axis=1)


def _edges(ei):
    e = ei.astype(jnp.int32)
    return e[0].reshape(NTILES, NC, K), e[1].reshape(NTILES, NC, K)


def _wr_weights(params, tag):
    out = []
    for rel in ("pg", "gp", "ps", "sp"):
        p = params[f"{tag}_{rel}"]
        out += [p["Wr"], p["b"].reshape(1, D)]
    return out


def _wl_weights(params, tag):
    return [params[f"{tag}_{rel}"]["Wl"] for rel in ("pg", "gp", "ps", "sp")]


def kernel(x_pfas_sites, x_gw_wells, x_sw_stations, params,
           edge_index_pg, edge_index_gp, edge_index_ps, edge_index_sp):
    x_p, x_g, x_s = x_pfas_sites, x_gw_wells, x_sw_stations

    spg, dpg = _edges(edge_index_pg)
    sgp, dgp = _edges(edge_index_gp)
    sps, dps = _edges(edge_index_ps)
    ssp, dsp = _edges(edge_index_sp)
    zrows = jnp.zeros((RPT0, DA), jnp.float32)

    bn = [params["bn_gw"]["g"].reshape(1, D), params["bn_gw"]["b"].reshape(1, D),
          params["bn_sw"]["g"].reshape(1, D), params["bn_sw"]["b"].reshape(1, D)]

    def seg(W, tab_p, tab_g, tab_s, zr):
        return _seg_sum(W, tab_p, tab_g, tab_p, tab_s,
                        spg, dpg, sgp, dgp, sps, dps, ssp, dsp, zr)

    xr1 = _dense_mmr(x_g, x_s, x_p, _wr_weights(params, "c1"))
    s1 = seg(DA, _augment(x_p), _augment(x_g), _augment(x_s), zrows)
    cnts = tuple(t[:, D:D + 1] for t in s1)
    h_g, h_s, h_p = _dense_ml_bn(s1, cnts, xr1, _wl_weights(params, "c1"),
                                 bn, ws=DA)

    xr2 = _dense_mmr(h_g, h_s, h_p, _wr_weights(params, "c2"))
    s2 = seg(D, h_p, h_g, h_s, zrows[:, :D])
    lin = [params["lin_gw"]["W"].reshape(1, D), params["lin_gw"]["b"].reshape(1, 1),
           params["lin_sw"]["W"].reshape(1, D), params["lin_sw"]["b"].reshape(1, 1),
           params["pr_gw"].reshape(1, 1), params["pr_sw"].reshape(1, 1)]
    out_g, out_s, o_p = _dense_ml_head(s2, cnts, xr2,
                                       _wl_weights(params, "c2"),
                                       bn, lin, ws=D)
    return out_g, out_s, o_p


# speedup vs baseline: 8.0699x; 1.0575x over previous
import functools

import jax
import jax.numpy as jnp
from jax import lax
from jax.experimental import pallas as pl
from jax.experimental.pallas import tpu as pltpu
from jax.experimental.pallas import tpu_sc as plsc

N = 10000
D = 128
DA = 144
E = 160000
NTILES = 16
EP = E // NTILES
K = 125
NC = EP // K
NCB = 10
RPT0 = 632
RPTL = N - (NTILES - 1) * RPT0


def _make_seg_sum(W):
    mesh = plsc.VectorSubcoreMesh(core_axis_name="c", subcore_axis_name="s")
    out_type = [jax.ShapeDtypeStruct((N, W), jnp.float32) for _ in range(4)]

    @functools.partial(
        pl.kernel,
        out_type=out_type,
        mesh=mesh,
        scratch_types=[
            pltpu.VMEM_SHARED((N, W), jnp.float32),
            pltpu.VMEM((NCB, K), jnp.int32),
            pltpu.VMEM((NCB, K), jnp.int32),
            pltpu.VMEM((2, K, W), jnp.float32),
            pltpu.SemaphoreType.DMA,
            pltpu.SemaphoreType.DMA,
        ],
        compiler_params=pltpu.CompilerParams(use_tc_tiling_on_sc=False),
    )
    def seg_sum(t0, t1, t2, t3,
                e0, e1, e2, e3, zrows,
                o0, o1, o2, o3,
                acc, sidx, didx, rbuf, gsem, ssem):
        c = lax.axis_index("c")
        s = lax.axis_index("s")

        def zero_slice():
            @pl.when(s < NTILES - 1)
            def _():
                pltpu.sync_copy(zrows, acc.at[pl.ds(s * RPT0, RPT0)])

            @pl.when(s == NTILES - 1)
            def _():
                pltpu.sync_copy(zrows.at[pl.ds(0, RPTL)],
                                acc.at[pl.ds((NTILES - 1) * RPT0, RPTL)])

        def process(tab, ed):
            se = ed.at[0]
            de = ed.at[1]
            pltpu.sync_copy(se.at[s, pl.ds(0, NCB)], sidx)
            pltpu.sync_copy(de.at[s, pl.ds(0, NCB)], didx)
            pltpu.async_copy(tab.at[sidx.at[0]], rbuf.at[0], gsem)

            def body(j, carry):
                b = lax.rem(j, 2)
                nb = 1 - b
                jm = lax.rem(j, NCB)
                jn = lax.rem(j + 1, NCB)

                @pl.when(jnp.logical_and(j >= 1, jm != 0))
                def _():
                    pltpu.make_async_copy(
                        rbuf.at[nb], acc.at[didx.at[lax.rem(j - 1, NCB)]],
                        ssem).wait()

                @pl.when(jnp.logical_and(j + 1 < NC, jn != 0))
                def _():
                    pltpu.async_copy(tab.at[sidx.at[jn]], rbuf.at[nb], gsem)

                pltpu.make_async_copy(
                    tab.at[sidx.at[jm]], rbuf.at[b], gsem).wait()
                pltpu.async_copy(rbuf.at[b], acc.at[didx.at[jm]], ssem,
                                 add=True)

                @pl.when(jnp.logical_and(j + 1 < NC, jn == 0))
                def _():
                    pltpu.make_async_copy(
                        rbuf.at[b], acc.at[didx.at[jm]], ssem).wait()
                    pltpu.sync_copy(se.at[s, pl.ds(j + 1, NCB)], sidx)
                    pltpu.sync_copy(de.at[s, pl.ds(j + 1, NCB)], didx)
                    pltpu.async_copy(tab.at[sidx.at[0]], rbuf.at[nb], gsem)
                return carry

            lax.fori_loop(0, NC, body, 0)
            pltpu.make_async_copy(
                rbuf.at[(NC - 1) % 2],
                acc.at[didx.at[(NC - 1) % NCB]], ssem).wait()

        def flush(out):
            @pl.when(s < NTILES - 1)
            def _():
                pltpu.sync_copy(acc.at[pl.ds(s * RPT0, RPT0)],
                                out.at[pl.ds(s * RPT0, RPT0)])

            @pl.when(s == NTILES - 1)
            def _():
                pltpu.sync_copy(acc.at[pl.ds((NTILES - 1) * RPT0, RPTL)],
                                out.at[pl.ds((NTILES - 1) * RPT0, RPTL)])

        def run_pair(tA, eA, oA, tB, eB, oB):
            zero_slice()
            plsc.subcore_barrier()
            process(tA, eA)
            plsc.subcore_barrier()
            flush(oA)
            zero_slice()
            plsc.subcore_barrier()
            process(tB, eB)
            plsc.subcore_barrier()
            flush(oB)

        @pl.when(c == 0)
        def _():
            run_pair(t0, e0, o0, t1, e1, o1)

        @pl.when(c == 1)
        def _():
            run_pair(t2, e2, o2, t3, e3, o3)

    return seg_sum


_seg_sum_cache = {}


def _seg_sum(W, *args):
    if W not in _seg_sum_cache:
        _seg_sum_cache[W] = _make_seg_sum(W)
    return _seg_sum_cache[W](*args)


BS = 2000
NB = N // BS


def _mmr_body(xg_ref, xs_ref, xp_ref,
              wrpg, bpg, wrgp, bgp, wrps, bps, wrsp, bsp,
              rg_ref, rs_ref, rp_ref):
    f32 = jnp.float32
    rg_ref[...] = jnp.dot(xg_ref[...], wrpg[...],
                          preferred_element_type=f32) + bpg[...]
    rs_ref[...] = jnp.dot(xs_ref[...], wrps[...],
                          preferred_element_type=f32) + bps[...]
    rp_ref[...] = (jnp.dot(xp_ref[...], wrgp[...],
                           preferred_element_type=f32) + bgp[...]
                   + jnp.dot(xp_ref[...], wrsp[...],
                             preferred_element_type=f32) + bsp[...])


def _dense_mmr(xg, xs, xp, w, interpret=False):
    outs = (jax.ShapeDtypeStruct((N, D), jnp.float32),) * 3
    row = pl.BlockSpec((BS, D), lambda i: (i, 0))
    in_specs = ([row] * 3
                + [pl.BlockSpec((D, D), lambda i: (0, 0)),
                   pl.BlockSpec((1, D), lambda i: (0, 0))] * 4)
    return pl.pallas_call(
        _mmr_body, grid=(NB,), out_shape=outs,
        in_specs=in_specs, out_specs=(row,) * 3,
        interpret=interpret)(xg, xs, xp, *w)


def _bn_stats(st):
    mg = st[0:1, :] / N
    vg = st[1:2, :] / N - mg * mg
    ms = st[2:3, :] / N
    vs = st[3:4, :] / N - ms * ms
    return mg, vg, ms, vs


def _ml_phase0(spg_ref, sgp_ref, sps_ref, ssp_ref,
               cpg_ref, cgp_ref, cps_ref, csp_ref,
               rg_ref, rs_ref, rp_ref,
               wlpg, wlgp, wlps, wlsp,
               hp_ref, pre_g, pre_s, sts, i):
    f32 = jnp.float32

    def ml(s_ref, c_ref, wl):
        cnt = c_ref[...][:, D:D + 1]
        mean = s_ref[...][:, :D] / jnp.maximum(cnt, 1.0)
        return jnp.dot(mean, wl[...], preferred_element_type=f32)

    hg = ml(spg_ref, cpg_ref, wlpg) + rg_ref[...]
    hs = ml(sps_ref, cps_ref, wlps) + rs_ref[...]
    hp_ref[...] = (ml(sgp_ref, cgp_ref, wlgp)
                   + ml(ssp_ref, csp_ref, wlsp) + rp_ref[...])
    pre_g[i] = hg
    pre_s[i] = hs

    z = jnp.zeros((1, D), f32)
    blk = jnp.concatenate(
        [jnp.sum(hg, axis=0, keepdims=True),
         jnp.sum(hg * hg, axis=0, keepdims=True),
         jnp.sum(hs, axis=0, keepdims=True),
         jnp.sum(hs * hs, axis=0, keepdims=True),
         z, z, z, z], axis=0)

    @pl.when(i == 0)
    def _():
        sts[...] = blk

    @pl.when(i > 0)
    def _():
        sts[...] = sts[...] + blk


def _ml_bn_body(spg_ref, sgp_ref, sps_ref, ssp_ref,
                rg_ref, rs_ref, rp_ref,
                wlpg, wlgp, wlps, wlsp,
                gg, gbg, gs, gbs,
                hg_ref, hs_ref, hp_ref,
                pre_g, pre_s, sts):
    ph = pl.program_id(0)
    i = pl.program_id(1)

    @pl.when(ph == 0)
    def _():
        _ml_phase0(spg_ref, sgp_ref, sps_ref, ssp_ref,
                   spg_ref, sgp_ref, sps_ref, ssp_ref,
                   rg_ref, rs_ref, rp_ref,
                   wlpg, wlgp, wlps, wlsp,
                   hp_ref, pre_g, pre_s, sts, i)

    @pl.when(ph == 1)
    def _():
        mg, vg, ms, vs = _bn_stats(sts[...])
        hg_ref[...] = jax.nn.relu(
            (pre_g[i] - mg) * jax.lax.rsqrt(vg + 1e-5) * gg[...] + gbg[...])
        hs_ref[...] = jax.nn.relu(
            (pre_s[i] - ms) * jax.lax.rsqrt(vs + 1e-5) * gs[...] + gbs[...])


def _ml_head_body(spg_ref, sgp_ref, sps_ref, ssp_ref,
                  cpg_ref, cgp_ref, cps_ref, csp_ref,
                  rg_ref, rs_ref, rp_ref,
                  wlpg, wlgp, wlps, wlsp,
                  gg, gbg, gs, gbs,
                  lwg, lbg, lws, lbs, ag, as_,
                  outg_ref, outs_ref, op_ref,
                  pre_g, pre_s, sts):
    ph = pl.program_id(0)
    i = pl.program_id(1)

    @pl.when(ph == 0)
    def _():
        _ml_phase0(spg_ref, sgp_ref, sps_ref, ssp_ref,
                   cpg_ref, cgp_ref, cps_ref, csp_ref,
                   rg_ref, rs_ref, rp_ref,
                   wlpg, wlgp, wlps, wlsp,
                   op_ref, pre_g, pre_s, sts, i)

    @pl.when(ph == 1)
    def _():
        mg, vg, ms, vs = _bn_stats(sts[...])
        og = jax.nn.relu(
            (pre_g[i] - mg) * jax.lax.rsqrt(vg + 1e-5) * gg[...] + gbg[...])
        os_ = jax.nn.relu(
            (pre_s[i] - ms) * jax.lax.rsqrt(vs + 1e-5) * gs[...] + gbs[...])

        def head(o, lw, lb, a):
            t = jnp.sum(o * lw[...], axis=1, keepdims=True) + lb[...]
            return jnp.where(t >= 0, t, a[...] * t)

        outg_ref[...] = head(og, lwg, lbg, ag)
        outs_ref[...] = head(os_, lws, lbs, as_)


def _p0_spec(cols):
    return pl.BlockSpec(
        (BS, cols), lambda ph, i: (jnp.where(ph == 0, i, 0), 0))


def _p1out_spec(cols):
    return pl.BlockSpec(
        (BS, cols), lambda ph, i: (jnp.where(ph == 0, 0, i), 0))


def _p0out_spec(cols):
    return pl.BlockSpec(
        (BS, cols), lambda ph, i: (jnp.where(ph == 0, i, NB - 1), 0))


def _pfull_spec(r, c):
    return pl.BlockSpec((r, c), lambda ph, i: (0, 0))


def _ml_scratch():
    return [pltpu.VMEM((NB, BS, D), jnp.float32),
            pltpu.VMEM((NB, BS, D), jnp.float32),
            pltpu.VMEM((8, D), jnp.float32)]


def _dense_ml_bn(s, r, wl, bn, interpret=False):
    outs = (jax.ShapeDtypeStruct((N, D), jnp.float32),) * 3
    in_specs = ([_p0_spec(DA)] * 4 + [_p0_spec(D)] * 3
                + [_pfull_spec(D, D)] * 4 + [_pfull_spec(1, D)] * 4)
    out_specs = (_p1out_spec(D), _p1out_spec(D), _p0out_spec(D))
    return pl.pallas_call(
        _ml_bn_body, grid=(2, NB), out_shape=outs,
        in_specs=in_specs, out_specs=out_specs,
        scratch_shapes=_ml_scratch(),
        interpret=interpret)(*s, *r, *wl, *bn)


def _dense_ml_head(s, cnts, r, wl, bn, lin, interpret=False):
    outs = (jax.ShapeDtypeStruct((N, 1), jnp.float32),
            jax.ShapeDtypeStruct((N, 1), jnp.float32),
            jax.ShapeDtypeStruct((N, D), jnp.float32))
    in_specs = ([_p0_spec(D)] * 4 + [_p0_spec(DA)] * 4 + [_p0_spec(D)] * 3
                + [_pfull_spec(D, D)] * 4 + [_pfull_spec(1, D)] * 4
                + [_pfull_spec(1, D), _pfull_spec(1, 1)] * 2
                + [_pfull_spec(1, 1)] * 2)
    out_specs = (_p1out_spec(1), _p1out_spec(1), _p0out_spec(D))
    return pl.pallas_call(
        _ml_head_body, grid=(2, NB), out_shape=outs,
        in_specs=in_specs, out_specs=out_specs,
        scratch_shapes=_ml_scratch(),
        interpret=interpret)(*s, *cnts, *r, *wl, *bn, *lin)


def _augment(x):
    ones = jnp.ones((x.shape[0], 1), x.dtype)
    zpad = jnp.zeros((x.shape[0], DA - D - 1), x.dtype)
    return jnp.concatenate([x, ones, zpad], axis=1)


def _edges(ei):
    return ei.astype(jnp.int32).reshape(2, NTILES, NC, K)


def _wr_weights(params, tag):
    out = []
    for rel in ("pg", "gp", "ps", "sp"):
        p = params[f"{tag}_{rel}"]
        out += [p["Wr"], p["b"].reshape(1, D)]
    return out


def _wl_weights(params, tag):
    return [params[f"{tag}_{rel}"]["Wl"] for rel in ("pg", "gp", "ps", "sp")]


def kernel(x_pfas_sites, x_gw_wells, x_sw_stations, params,
           edge_index_pg, edge_index_gp, edge_index_ps, edge_index_sp):
    x_p, x_g, x_s = x_pfas_sites, x_gw_wells, x_sw_stations

    e_pg = _edges(edge_index_pg)
    e_gp = _edges(edge_index_gp)
    e_ps = _edges(edge_index_ps)
    e_sp = _edges(edge_index_sp)
    zrows = jnp.zeros((RPT0, DA), jnp.float32)

    bn = [params["bn_gw"]["g"].reshape(1, D), params["bn_gw"]["b"].reshape(1, D),
          params["bn_sw"]["g"].reshape(1, D), params["bn_sw"]["b"].reshape(1, D)]

    def seg(W, tab_p, tab_g, tab_s, zr):
        return _seg_sum(W, tab_p, tab_g, tab_p, tab_s,
                        e_pg, e_gp, e_ps, e_sp, zr)

    xr1 = _dense_mmr(x_g, x_s, x_p, _wr_weights(params, "c1"))
    s1 = seg(DA, _augment(x_p), _augment(x_g), _augment(x_s), zrows)
    h_g, h_s, h_p = _dense_ml_bn(s1, xr1, _wl_weights(params, "c1"), bn)

    xr2 = _dense_mmr(h_g, h_s, h_p, _wr_weights(params, "c2"))
    s2 = seg(D, h_p, h_g, h_s, zrows[:, :D])
    lin = [params["lin_gw"]["W"].reshape(1, D), params["lin_gw"]["b"].reshape(1, 1),
           params["lin_sw"]["W"].reshape(1, D), params["lin_sw"]["b"].reshape(1, 1),
           params["pr_gw"].reshape(1, 1), params["pr_sw"].reshape(1, 1)]
    out_g, out_s, o_p = _dense_ml_head(s2, s1, xr2,
                                       _wl_weights(params, "c2"),
                                       bn, lin)
    return out_g, out_s, o_p


# speedup vs baseline: 8.4312x; 1.0448x over previous
import functools

import jax
import jax.numpy as jnp
from jax import lax
from jax.experimental import pallas as pl
from jax.experimental.pallas import tpu as pltpu
from jax.experimental.pallas import tpu_sc as plsc

N = 10000
D = 128
DA = 144
E = 160000
NTILES = 16
EP = E // NTILES
K = 125
NC = EP // K
NCB = 10
RPT0 = 632
RPTL = N - (NTILES - 1) * RPT0


def _make_seg_sum(W):
    mesh = plsc.VectorSubcoreMesh(core_axis_name="c", subcore_axis_name="s")
    out_type = [jax.ShapeDtypeStruct((N, D), jnp.float32) for _ in range(4)]
    if W == DA:
        out_type += [jax.ShapeDtypeStruct((N, DA - D), jnp.float32)
                     for _ in range(4)]

    @functools.partial(
        pl.kernel,
        out_type=out_type,
        mesh=mesh,
        scratch_types=[
            pltpu.VMEM_SHARED((N, W), jnp.float32),
            pltpu.VMEM((NCB, K), jnp.int32),
            pltpu.VMEM((NCB, K), jnp.int32),
            pltpu.VMEM((2, K, W), jnp.float32),
            pltpu.SemaphoreType.DMA,
            pltpu.SemaphoreType.DMA,
        ],
        compiler_params=pltpu.CompilerParams(use_tc_tiling_on_sc=False),
    )
    def seg_sum(t0, t1, t2, t3,
                e0, e1, e2, e3, zrows,
                *outs_and_scratch):
        if W == DA:
            (o0, o1, o2, o3, q0, q1, q2, q3,
             acc, sidx, didx, rbuf, gsem, ssem) = outs_and_scratch
            cnt_outs = (q0, q1, q2, q3)
        else:
            (o0, o1, o2, o3,
             acc, sidx, didx, rbuf, gsem, ssem) = outs_and_scratch
            cnt_outs = (None, None, None, None)
        c = lax.axis_index("c")
        s = lax.axis_index("s")

        def zero_slice():
            @pl.when(s < NTILES - 1)
            def _():
                pltpu.sync_copy(zrows, acc.at[pl.ds(s * RPT0, RPT0)])

            @pl.when(s == NTILES - 1)
            def _():
                pltpu.sync_copy(zrows.at[pl.ds(0, RPTL)],
                                acc.at[pl.ds((NTILES - 1) * RPT0, RPTL)])

        def process(tab, ed):
            se = ed.at[0]
            de = ed.at[1]
            pltpu.sync_copy(se.at[s, pl.ds(0, NCB)], sidx)
            pltpu.sync_copy(de.at[s, pl.ds(0, NCB)], didx)
            pltpu.async_copy(tab.at[sidx.at[0]], rbuf.at[0], gsem)

            def body(j, carry):
                b = lax.rem(j, 2)
                nb = 1 - b
                jm = lax.rem(j, NCB)
                jn = lax.rem(j + 1, NCB)

                @pl.when(jnp.logical_and(j >= 1, jm != 0))
                def _():
                    pltpu.make_async_copy(
                        rbuf.at[nb], acc.at[didx.at[lax.rem(j - 1, NCB)]],
                        ssem).wait()

                @pl.when(jnp.logical_and(j + 1 < NC, jn != 0))
                def _():
                    pltpu.async_copy(tab.at[sidx.at[jn]], rbuf.at[nb], gsem)

                pltpu.make_async_copy(
                    tab.at[sidx.at[jm]], rbuf.at[b], gsem).wait()
                pltpu.async_copy(rbuf.at[b], acc.at[didx.at[jm]], ssem,
                                 add=True)

                @pl.when(jnp.logical_and(j + 1 < NC, jn == 0))
                def _():
                    pltpu.make_async_copy(
                        rbuf.at[b], acc.at[didx.at[jm]], ssem).wait()
                    pltpu.sync_copy(se.at[s, pl.ds(j + 1, NCB)], sidx)
                    pltpu.sync_copy(de.at[s, pl.ds(j + 1, NCB)], didx)
                    pltpu.async_copy(tab.at[sidx.at[0]], rbuf.at[nb], gsem)
                return carry

            lax.fori_loop(0, NC, body, 0)
            pltpu.make_async_copy(
                rbuf.at[(NC - 1) % 2],
                acc.at[didx.at[(NC - 1) % NCB]], ssem).wait()

        def flush(out, qout):
            def emit(r0, nr):
                if W == DA:
                    pltpu.sync_copy(acc.at[pl.ds(r0, nr), pl.ds(0, D)],
                                    out.at[pl.ds(r0, nr)])
                    pltpu.sync_copy(acc.at[pl.ds(r0, nr), pl.ds(D, DA - D)],
                                    qout.at[pl.ds(r0, nr)])
                else:
                    pltpu.sync_copy(acc.at[pl.ds(r0, nr)],
                                    out.at[pl.ds(r0, nr)])

            @pl.when(s < NTILES - 1)
            def _():
                emit(s * RPT0, RPT0)

            @pl.when(s == NTILES - 1)
            def _():
                emit((NTILES - 1) * RPT0, RPTL)

        def run_pair(tA, eA, oA, qA, tB, eB, oB, qB):
            zero_slice()
            plsc.subcore_barrier()
            process(tA, eA)
            plsc.subcore_barrier()
            flush(oA, qA)
            zero_slice()
            plsc.subcore_barrier()
            process(tB, eB)
            plsc.subcore_barrier()
            flush(oB, qB)

        @pl.when(c == 0)
        def _():
            run_pair(t0, e0, o0, cnt_outs[0], t1, e1, o1, cnt_outs[1])

        @pl.when(c == 1)
        def _():
            run_pair(t2, e2, o2, cnt_outs[2], t3, e3, o3, cnt_outs[3])

    return seg_sum


_seg_sum_cache = {}


def _seg_sum(W, *args):
    if W not in _seg_sum_cache:
        _seg_sum_cache[W] = _make_seg_sum(W)
    return _seg_sum_cache[W](*args)


BS = 2000
NB = N // BS


def _mmr_body(xg_ref, xs_ref, xp_ref,
              wrpg, bpg, wrgp, bgp, wrps, bps, wrsp, bsp,
              rg_ref, rs_ref, rp_ref):
    f32 = jnp.float32
    rg_ref[...] = jnp.dot(xg_ref[...], wrpg[...],
                          preferred_element_type=f32) + bpg[...]
    rs_ref[...] = jnp.dot(xs_ref[...], wrps[...],
                          preferred_element_type=f32) + bps[...]
    rp_ref[...] = (jnp.dot(xp_ref[...], wrgp[...],
                           preferred_element_type=f32) + bgp[...]
                   + jnp.dot(xp_ref[...], wrsp[...],
                             preferred_element_type=f32) + bsp[...])


def _dense_mmr(xg, xs, xp, w, interpret=False):
    outs = (jax.ShapeDtypeStruct((N, D), jnp.float32),) * 3
    row = pl.BlockSpec((BS, D), lambda i: (i, 0))
    in_specs = ([row] * 3
                + [pl.BlockSpec((D, D), lambda i: (0, 0)),
                   pl.BlockSpec((1, D), lambda i: (0, 0))] * 4)
    return pl.pallas_call(
        _mmr_body, grid=(NB,), out_shape=outs,
        in_specs=in_specs, out_specs=(row,) * 3,
        interpret=interpret)(xg, xs, xp, *w)


def _bn_stats(st):
    mg = st[0:1, :] / N
    vg = st[1:2, :] / N - mg * mg
    ms = st[2:3, :] / N
    vs = st[3:4, :] / N - ms * ms
    return mg, vg, ms, vs


def _ml_phase0(spg_ref, sgp_ref, sps_ref, ssp_ref,
               cpg_ref, cgp_ref, cps_ref, csp_ref,
               rg_ref, rs_ref, rp_ref,
               wlpg, wlgp, wlps, wlsp,
               hp_ref, pre_g, pre_s, sts, i):
    f32 = jnp.float32

    def ml(s_ref, c_ref, wl):
        cnt = c_ref[...][:, :1]
        mean = s_ref[...] / jnp.maximum(cnt, 1.0)
        return jnp.dot(mean, wl[...], preferred_element_type=f32)

    hg = ml(spg_ref, cpg_ref, wlpg) + rg_ref[...]
    hs = ml(sps_ref, cps_ref, wlps) + rs_ref[...]
    hp_ref[...] = (ml(sgp_ref, cgp_ref, wlgp)
                   + ml(ssp_ref, csp_ref, wlsp) + rp_ref[...])
    pre_g[i] = hg
    pre_s[i] = hs

    z = jnp.zeros((1, D), f32)
    blk = jnp.concatenate(
        [jnp.sum(hg, axis=0, keepdims=True),
         jnp.sum(hg * hg, axis=0, keepdims=True),
         jnp.sum(hs, axis=0, keepdims=True),
         jnp.sum(hs * hs, axis=0, keepdims=True),
         z, z, z, z], axis=0)

    @pl.when(i == 0)
    def _():
        sts[...] = blk

    @pl.when(i > 0)
    def _():
        sts[...] = sts[...] + blk


def _ml_bn_body(spg_ref, sgp_ref, sps_ref, ssp_ref,
                cpg_ref, cgp_ref, cps_ref, csp_ref,
                rg_ref, rs_ref, rp_ref,
                wlpg, wlgp, wlps, wlsp,
                gg, gbg, gs, gbs,
                hg_ref, hs_ref, hp_ref,
                pre_g, pre_s, sts):
    ph = pl.program_id(0)
    i = pl.program_id(1)

    @pl.when(ph == 0)
    def _():
        _ml_phase0(spg_ref, sgp_ref, sps_ref, ssp_ref,
                   cpg_ref, cgp_ref, cps_ref, csp_ref,
                   rg_ref, rs_ref, rp_ref,
                   wlpg, wlgp, wlps, wlsp,
                   hp_ref, pre_g, pre_s, sts, i)

    @pl.when(ph == 1)
    def _():
        mg, vg, ms, vs = _bn_stats(sts[...])
        hg_ref[...] = jax.nn.relu(
            (pre_g[i] - mg) * jax.lax.rsqrt(vg + 1e-5) * gg[...] + gbg[...])
        hs_ref[...] = jax.nn.relu(
            (pre_s[i] - ms) * jax.lax.rsqrt(vs + 1e-5) * gs[...] + gbs[...])


def _ml_head_body(spg_ref, sgp_ref, sps_ref, ssp_ref,
                  cpg_ref, cgp_ref, cps_ref, csp_ref,
                  rg_ref, rs_ref, rp_ref,
                  wlpg, wlgp, wlps, wlsp,
                  gg, gbg, gs, gbs,
                  lwg, lbg, lws, lbs, ag, as_,
                  outg_ref, outs_ref, op_ref,
                  pre_g, pre_s, sts):
    ph = pl.program_id(0)
    i = pl.program_id(1)

    @pl.when(ph == 0)
    def _():
        _ml_phase0(spg_ref, sgp_ref, sps_ref, ssp_ref,
                   cpg_ref, cgp_ref, cps_ref, csp_ref,
                   rg_ref, rs_ref, rp_ref,
                   wlpg, wlgp, wlps, wlsp,
                   op_ref, pre_g, pre_s, sts, i)

    @pl.when(ph == 1)
    def _():
        mg, vg, ms, vs = _bn_stats(sts[...])
        og = jax.nn.relu(
            (pre_g[i] - mg) * jax.lax.rsqrt(vg + 1e-5) * gg[...] + gbg[...])
        os_ = jax.nn.relu(
            (pre_s[i] - ms) * jax.lax.rsqrt(vs + 1e-5) * gs[...] + gbs[...])

        def head(o, lw, lb, a):
            t = jnp.sum(o * lw[...], axis=1, keepdims=True) + lb[...]
            return jnp.where(t >= 0, t, a[...] * t)

        outg_ref[...] = head(og, lwg, lbg, ag)
        outs_ref[...] = head(os_, lws, lbs, as_)


def _p0_spec(cols):
    return pl.BlockSpec(
        (BS, cols), lambda ph, i: (jnp.where(ph == 0, i, 0), 0))


def _p1out_spec(cols):
    return pl.BlockSpec(
        (BS, cols), lambda ph, i: (jnp.where(ph == 0, 0, i), 0))


def _p0out_spec(cols):
    return pl.BlockSpec(
        (BS, cols), lambda ph, i: (jnp.where(ph == 0, i, NB - 1), 0))


def _pfull_spec(r, c):
    return pl.BlockSpec((r, c), lambda ph, i: (0, 0))


def _ml_scratch():
    return [pltpu.VMEM((NB, BS, D), jnp.float32),
            pltpu.VMEM((NB, BS, D), jnp.float32),
            pltpu.VMEM((8, D), jnp.float32)]


def _dense_ml_bn(s, cnts, r, wl, bn, interpret=False):
    outs = (jax.ShapeDtypeStruct((N, D), jnp.float32),) * 3
    in_specs = ([_p0_spec(D)] * 4 + [_p0_spec(DA - D)] * 4 + [_p0_spec(D)] * 3
                + [_pfull_spec(D, D)] * 4 + [_pfull_spec(1, D)] * 4)
    out_specs = (_p1out_spec(D), _p1out_spec(D), _p0out_spec(D))
    return pl.pallas_call(
        _ml_bn_body, grid=(2, NB), out_shape=outs,
        in_specs=in_specs, out_specs=out_specs,
        scratch_shapes=_ml_scratch(),
        interpret=interpret)(*s, *cnts, *r, *wl, *bn)


def _dense_ml_head(s, cnts, r, wl, bn, lin, interpret=False):
    outs = (jax.ShapeDtypeStruct((N, 1), jnp.float32),
            jax.ShapeDtypeStruct((N, 1), jnp.float32),
            jax.ShapeDtypeStruct((N, D), jnp.float32))
    in_specs = ([_p0_spec(D)] * 4 + [_p0_spec(DA - D)] * 4 + [_p0_spec(D)] * 3
                + [_pfull_spec(D, D)] * 4 + [_pfull_spec(1, D)] * 4
                + [_pfull_spec(1, D), _pfull_spec(1, 1)] * 2
                + [_pfull_spec(1, 1)] * 2)
    out_specs = (_p1out_spec(1), _p1out_spec(1), _p0out_spec(D))
    return pl.pallas_call(
        _ml_head_body, grid=(2, NB), out_shape=outs,
        in_specs=in_specs, out_specs=out_specs,
        scratch_shapes=_ml_scratch(),
        interpret=interpret)(*s, *cnts, *r, *wl, *bn, *lin)


def _augment(x):
    ones = jnp.ones((x.shape[0], 1), x.dtype)
    zpad = jnp.zeros((x.shape[0], DA - D - 1), x.dtype)
    return jnp.concatenate([x, ones, zpad], axis=1)


def _edges(ei):
    return ei.astype(jnp.int32).reshape(2, NTILES, NC, K)


def _wr_weights(params, tag):
    out = []
    for rel in ("pg", "gp", "ps", "sp"):
        p = params[f"{tag}_{rel}"]
        out += [p["Wr"], p["b"].reshape(1, D)]
    return out


def _wl_weights(params, tag):
    return [params[f"{tag}_{rel}"]["Wl"] for rel in ("pg", "gp", "ps", "sp")]


def kernel(x_pfas_sites, x_gw_wells, x_sw_stations, params,
           edge_index_pg, edge_index_gp, edge_index_ps, edge_index_sp):
    x_p, x_g, x_s = x_pfas_sites, x_gw_wells, x_sw_stations

    e_pg = _edges(edge_index_pg)
    e_gp = _edges(edge_index_gp)
    e_ps = _edges(edge_index_ps)
    e_sp = _edges(edge_index_sp)
    zrows = jnp.zeros((RPT0, DA), jnp.float32)

    bn = [params["bn_gw"]["g"].reshape(1, D), params["bn_gw"]["b"].reshape(1, D),
          params["bn_sw"]["g"].reshape(1, D), params["bn_sw"]["b"].reshape(1, D)]

    def seg(W, tab_p, tab_g, tab_s, zr):
        return _seg_sum(W, tab_p, tab_g, tab_p, tab_s,
                        e_pg, e_gp, e_ps, e_sp, zr)

    xr1 = _dense_mmr(x_g, x_s, x_p, _wr_weights(params, "c1"))
    r1 = seg(DA, _augment(x_p), _augment(x_g), _augment(x_s), zrows)
    s1, cnts = r1[:4], r1[4:]
    h_g, h_s, h_p = _dense_ml_bn(s1, cnts, xr1, _wl_weights(params, "c1"), bn)

    xr2 = _dense_mmr(h_g, h_s, h_p, _wr_weights(params, "c2"))
    s2 = seg(D, h_p, h_g, h_s, zrows[:, :D])
    lin = [params["lin_gw"]["W"].reshape(1, D), params["lin_gw"]["b"].reshape(1, 1),
           params["lin_sw"]["W"].reshape(1, D), params["lin_sw"]["b"].reshape(1, 1),
           params["pr_gw"].reshape(1, 1), params["pr_sw"].reshape(1, 1)]
    out_g, out_s, o_p = _dense_ml_head(s2, cnts, xr2,
                                       _wl_weights(params, "c2"),
                                       bn, lin)
    return out_g, out_s, o_p
